# baseline (speedup 1.0000x reference)
import functools

import jax
import jax.numpy as jnp
from jax import lax
from jax.experimental import pallas as pl
from jax.experimental.pallas import tpu as pltpu
from jax.experimental.pallas import tpu_sc as plsc

B = 16384
ED = 64
CF = 64
PF = 32
NW = 32
BPW = B // NW


def _sc_gather(ct_hbm, pt_hbm, cid_hbm, pid_hbm, ce_hbm, pe_hbm,
               cidx_v, pidx_v, crow_v, prow_v, sem_c, sem_p):
    wid = lax.axis_index("s") * 2 + lax.axis_index("c")
    base = wid * BPW
    pltpu.sync_copy(cid_hbm.at[pl.ds(base, BPW)], cidx_v)
    pltpu.sync_copy(pid_hbm.at[pl.ds(base, BPW)], pidx_v)
    cpy_c = pltpu.async_copy(ct_hbm.at[cidx_v], crow_v, sem_c)
    cpy_p = pltpu.async_copy(pt_hbm.at[pidx_v], prow_v, sem_p)
    cpy_c.wait()
    pltpu.sync_copy(crow_v, ce_hbm.at[pl.ds(base, BPW)])
    cpy_p.wait()
    pltpu.sync_copy(prow_v, pe_hbm.at[pl.ds(base, BPW)])


def _mlp_body(ce, pe, cf, pf, w1c, w1p, w1cf, w1pf, b1, w2, b2, w3, b3, w4, b4,
              out_ref):
    x = jnp.dot(ce[...], w1c[...], preferred_element_type=jnp.float32)
    x += jnp.dot(pe[...], w1p[...], preferred_element_type=jnp.float32)
    x += jnp.dot(cf[...], w1cf[...], preferred_element_type=jnp.float32)
    x += jnp.dot(pf[...], w1pf[...], preferred_element_type=jnp.float32)
    h = jax.nn.relu(x + b1[...])
    h = jax.nn.relu(jnp.dot(h, w2[...], preferred_element_type=jnp.float32)
                    + b2[...])
    h = jax.nn.relu(jnp.dot(h, w3[...], preferred_element_type=jnp.float32)
                    + b3[...])
    o = jnp.dot(h, w4[...], preferred_element_type=jnp.float32) + b4[...]
    out_ref[...] = jax.nn.sigmoid(o)


def kernel(customer_ids, product_ids, customer_features, product_features,
           customer_table, product_table,
           W1, b1, g1, beta1, W2, b2, g2, beta2, W3, b3, g3, beta3, W4, b4):
    mesh = plsc.VectorSubcoreMesh(core_axis_name="c", subcore_axis_name="s")
    gather = pl.kernel(
        _sc_gather,
        out_type=(jax.ShapeDtypeStruct((B, ED), jnp.float32),
                  jax.ShapeDtypeStruct((B, ED), jnp.float32)),
        mesh=mesh,
        scratch_types=[
            pltpu.VMEM((BPW,), jnp.int32),
            pltpu.VMEM((BPW,), jnp.int32),
            pltpu.VMEM((BPW, ED), jnp.float32),
            pltpu.VMEM((BPW, ED), jnp.float32),
            pltpu.SemaphoreType.DMA,
            pltpu.SemaphoreType.DMA,
        ],
        compiler_params=pltpu.CompilerParams(use_tc_tiling_on_sc=False),
    )
    ce, pe = gather(customer_table, product_table,
                    customer_ids.astype(jnp.int32),
                    product_ids.astype(jnp.int32))

    inv = 1.0 / jnp.sqrt(1.0 + 1e-5)
    s1 = g1 * inv
    s2 = g2 * inv
    s3 = g3 * inv
    w1f = W1 * s1[:, None]
    b1f = (b1 * s1 + beta1).reshape(1, -1)
    w2f = (W2 * s2[:, None]).T
    b2f = (b2 * s2 + beta2).reshape(1, -1)
    w3f = (W3 * s3[:, None]).T
    b3f = (b3 * s3 + beta3).reshape(1, -1)
    w4t = W4.T
    b4r = b4.reshape(1, -1)
    w1c = w1f[:, :ED].T
    w1p = w1f[:, ED:2 * ED].T
    w1cf = w1f[:, 2 * ED:2 * ED + CF].T
    w1pf = w1f[:, 2 * ED + CF:].T

    BM = 2048
    grid = B // BM
    row = lambda i: (i, 0)
    full = lambda i: (0, 0)
    out = pl.pallas_call(
        _mlp_body,
        grid=(grid,),
        in_specs=[
            pl.BlockSpec((BM, ED), row),
            pl.BlockSpec((BM, ED), row),
            pl.BlockSpec((BM, CF), row),
            pl.BlockSpec((BM, PF), row),
            pl.BlockSpec((ED, 256), full),
            pl.BlockSpec((ED, 256), full),
            pl.BlockSpec((CF, 256), full),
            pl.BlockSpec((PF, 256), full),
            pl.BlockSpec((1, 256), full),
            pl.BlockSpec((256, 128), full),
            pl.BlockSpec((1, 128), full),
            pl.BlockSpec((128, 64), full),
            pl.BlockSpec((1, 64), full),
            pl.BlockSpec((64, 1), full),
            pl.BlockSpec((1, 1), full),
        ],
        out_specs=pl.BlockSpec((BM, 1), row),
        out_shape=jax.ShapeDtypeStruct((B, 1), jnp.float32),
    )(ce, pe, customer_features, product_features,
      w1c, w1p, w1cf, w1pf, b1f, w2f, b2f, w3f, b3f, w4t, b4r)
    return out
